# Initial kernel scaffold; baseline (speedup 1.0000x reference)
import jax
import jax.numpy as jnp
from jax.experimental import pallas as pl


def kernel(edge_index, node_features, W1, b1, W2, b2):
    raise NotImplementedError("write your pallas kernel here")


# speedup vs baseline: 13.3088x; 13.3088x over previous
import functools

import jax
import jax.numpy as jnp
from jax import lax
from jax.experimental import pallas as pl
from jax.experimental.pallas import tpu as pltpu
from jax.experimental.pallas import tpu_sc as plsc

N = 10000
D = 128
E = 320000

NC = 2
NS = 16
NW = NC * NS
EPW = E // NW
CHUNK = 80
NCHUNK = EPW // CHUNK
NP = 10240
DEG_PT = NP // NS
ROWS_PT = NP // NS
ZROWS = 128

_sc_mesh = plsc.VectorSubcoreMesh(core_axis_name="c", subcore_axis_name="s")


@functools.partial(
    pl.kernel,
    out_type=jax.ShapeDtypeStruct((NC, NP), jnp.float32),
    mesh=_sc_mesh,
    scratch_types=[
        pltpu.VMEM((CHUNK,), jnp.int32),
        pltpu.VMEM((CHUNK,), jnp.float32),
        pltpu.VMEM((DEG_PT,), jnp.float32),
        pltpu.VMEM_SHARED((NP,), jnp.float32),
        pltpu.SemaphoreType.DMA,
    ],
)
def _sc_degree(dst_hbm, out_hbm, idx_v, ones_v, zeros_v, deg_sh, sem):
    cid = lax.axis_index("c")
    sid = lax.axis_index("s")
    wid = sid * NC + cid

    for j in range(CHUNK // 16):
        ones_v[pl.ds(j * 16, 16)] = jnp.ones((16,), jnp.float32)
    for j in range(DEG_PT // 16):
        zeros_v[pl.ds(j * 16, 16)] = jnp.zeros((16,), jnp.float32)
    pltpu.sync_copy(zeros_v, deg_sh.at[pl.ds(sid * DEG_PT, DEG_PT)])
    plsc.subcore_barrier()

    def body(k, _):
        base = wid * EPW + k * CHUNK
        pltpu.sync_copy(dst_hbm.at[pl.ds(base, CHUNK)], idx_v)
        pltpu.sync_copy(ones_v, deg_sh.at[idx_v], add=True)
        return 0

    lax.fori_loop(0, NCHUNK, body, 0)
    plsc.subcore_barrier()
    pltpu.sync_copy(deg_sh.at[pl.ds(sid * DEG_PT, DEG_PT)],
                    out_hbm.at[cid, pl.ds(sid * DEG_PT, DEG_PT)])


@functools.partial(
    pl.kernel,
    out_type=jax.ShapeDtypeStruct((NC, NP, D), jnp.float32),
    mesh=_sc_mesh,
    scratch_types=[
        pltpu.VMEM((CHUNK,), jnp.int32),
        pltpu.VMEM((CHUNK,), jnp.int32),
        pltpu.VMEM((CHUNK, D), jnp.float32),
        pltpu.VMEM((ZROWS, D), jnp.float32),
        pltpu.VMEM_SHARED((NP, D), jnp.float32),
        pltpu.SemaphoreType.DMA,
    ],
)
def _sc_agg(g_hbm, src_hbm, dst_hbm, out_hbm, sidx, didx, rows, zbuf, acc_sh,
            sem):
    cid = lax.axis_index("c")
    sid = lax.axis_index("s")
    wid = sid * NC + cid

    def zero_row(r, _):
        for j in range(D // 16):
            zbuf[r, pl.ds(j * 16, 16)] = jnp.zeros((16,), jnp.float32)
        return 0

    lax.fori_loop(0, ZROWS, zero_row, 0)
    for i in range(ROWS_PT // ZROWS):
        pltpu.sync_copy(zbuf, acc_sh.at[pl.ds(sid * ROWS_PT + i * ZROWS, ZROWS)])
    plsc.subcore_barrier()

    def body(k, _):
        base = wid * EPW + k * CHUNK
        pltpu.sync_copy(src_hbm.at[pl.ds(base, CHUNK)], sidx)
        pltpu.sync_copy(dst_hbm.at[pl.ds(base, CHUNK)], didx)
        pltpu.async_copy(g_hbm.at[sidx], rows, sem).wait()
        pltpu.sync_copy(rows, acc_sh.at[didx], add=True)
        return 0

    lax.fori_loop(0, NCHUNK, body, 0)
    plsc.subcore_barrier()
    pltpu.sync_copy(acc_sh.at[pl.ds(sid * ROWS_PT, ROWS_PT)],
                    out_hbm.at[cid, pl.ds(sid * ROWS_PT, ROWS_PT)])


BLK = 1000


def _gemm_scale_body(x_ref, w_ref, dinv_ref, g_ref):
    h = jnp.dot(x_ref[...], w_ref[...], preferred_element_type=jnp.float32)
    g_ref[...] = h * dinv_ref[...]


def _tc_gemm_scale(x, w, dinv):
    return pl.pallas_call(
        _gemm_scale_body,
        grid=(N // BLK,),
        in_specs=[
            pl.BlockSpec((BLK, D), lambda i: (i, 0)),
            pl.BlockSpec((D, D), lambda i: (0, 0)),
            pl.BlockSpec((BLK, 1), lambda i: (i, 0)),
        ],
        out_specs=pl.BlockSpec((BLK, D), lambda i: (i, 0)),
        out_shape=jax.ShapeDtypeStruct((N, D), jnp.float32),
    )(x, w, dinv)


def _mid_body(p_ref, g_ref, dinv_ref, b_ref, w_ref, o_ref):
    acc = p_ref[0] + p_ref[1] + g_ref[...]
    x2 = jnp.maximum(acc * dinv_ref[...] + b_ref[...], 0.0)
    h2 = jnp.dot(x2, w_ref[...], preferred_element_type=jnp.float32)
    o_ref[...] = h2 * dinv_ref[...]


def _tc_mid(parts, g, dinv, b, w):
    return pl.pallas_call(
        _mid_body,
        grid=(N // BLK,),
        in_specs=[
            pl.BlockSpec((NC, BLK, D), lambda i: (0, i, 0)),
            pl.BlockSpec((BLK, D), lambda i: (i, 0)),
            pl.BlockSpec((BLK, 1), lambda i: (i, 0)),
            pl.BlockSpec((1, D), lambda i: (0, 0)),
            pl.BlockSpec((D, D), lambda i: (0, 0)),
        ],
        out_specs=pl.BlockSpec((BLK, D), lambda i: (i, 0)),
        out_shape=jax.ShapeDtypeStruct((N, D), jnp.float32),
    )(parts, g, dinv, b, w)


def _final_body(p_ref, g_ref, dinv_ref, b_ref, o_ref):
    acc = p_ref[0] + p_ref[1] + g_ref[...]
    o_ref[...] = acc * dinv_ref[...] + b_ref[...]


def _tc_final(parts, g, dinv, b):
    return pl.pallas_call(
        _final_body,
        grid=(N // BLK,),
        in_specs=[
            pl.BlockSpec((NC, BLK, D), lambda i: (0, i, 0)),
            pl.BlockSpec((BLK, D), lambda i: (i, 0)),
            pl.BlockSpec((BLK, 1), lambda i: (i, 0)),
            pl.BlockSpec((1, D), lambda i: (0, 0)),
        ],
        out_specs=pl.BlockSpec((BLK, D), lambda i: (i, 0)),
        out_shape=jax.ShapeDtypeStruct((N, D), jnp.float32),
    )(parts, g, dinv, b)


def kernel(edge_index, node_features, W1, b1, W2, b2):
    src = edge_index[0]
    dst = edge_index[1]

    degp = _sc_degree(dst)
    deg = degp[0, :N] + degp[1, :N] + 1.0
    dinv = lax.rsqrt(deg)[:, None]

    g1 = _tc_gemm_scale(node_features, W1, dinv)
    p1 = _sc_agg(g1, src, dst)
    g2 = _tc_mid(p1, g1, dinv, b1.reshape(1, D), W2)
    p2 = _sc_agg(g2, src, dst)
    return _tc_final(p2, g2, dinv, b2.reshape(1, D))


# speedup vs baseline: 31.7132x; 2.3829x over previous
import functools

import jax
import jax.numpy as jnp
from jax import lax
from jax.experimental import pallas as pl
from jax.experimental.pallas import tpu as pltpu
from jax.experimental.pallas import tpu_sc as plsc

N = 10000
D = 128
E = 320000

NC = 2
NS = 16
NW = NC * NS
EPW = E // NW
CHUNK = 80
NCHUNK = EPW // CHUNK
NP = 10240
DEG_PT = NP // NS
ROWS_PT = NP // NS
ZROWS = 128

_sc_mesh = plsc.VectorSubcoreMesh(core_axis_name="c", subcore_axis_name="s")


@functools.partial(
    pl.kernel,
    out_type=jax.ShapeDtypeStruct((NC, NP), jnp.float32),
    mesh=_sc_mesh,
    scratch_types=[
        pltpu.VMEM((NCHUNK, CHUNK), jnp.int32),
        pltpu.VMEM((CHUNK,), jnp.float32),
        pltpu.VMEM((DEG_PT,), jnp.float32),
        pltpu.VMEM_SHARED((NP,), jnp.float32),
        pltpu.SemaphoreType.DMA,
    ],
)
def _sc_degree(dst_hbm, out_hbm, idx_all, ones_v, zeros_v, deg_sh, sem):
    cid = lax.axis_index("c")
    sid = lax.axis_index("s")
    wid = sid * NC + cid

    pltpu.async_copy(dst_hbm.at[wid], idx_all, sem)
    for j in range(CHUNK // 16):
        ones_v[pl.ds(j * 16, 16)] = jnp.ones((16,), jnp.float32)
    for j in range(DEG_PT // 16):
        zeros_v[pl.ds(j * 16, 16)] = jnp.zeros((16,), jnp.float32)
    pltpu.sync_copy(zeros_v, deg_sh.at[pl.ds(sid * DEG_PT, DEG_PT)])
    pltpu.make_async_copy(dst_hbm.at[wid], idx_all, sem).wait()
    plsc.subcore_barrier()

    def body(k, _):
        pltpu.sync_copy(ones_v, deg_sh.at[idx_all.at[k]], add=True)
        return 0

    lax.fori_loop(0, NCHUNK, body, 0)
    plsc.subcore_barrier()
    pltpu.sync_copy(deg_sh.at[pl.ds(sid * DEG_PT, DEG_PT)],
                    out_hbm.at[cid, pl.ds(sid * DEG_PT, DEG_PT)])


@functools.partial(
    pl.kernel,
    out_type=jax.ShapeDtypeStruct((NC, NP, D), jnp.float32),
    mesh=_sc_mesh,
    scratch_types=[
        pltpu.VMEM((EPW,), jnp.int32),
        pltpu.VMEM((NCHUNK, CHUNK), jnp.int32),
        pltpu.VMEM((CHUNK, D), jnp.float32),
        pltpu.VMEM((CHUNK, D), jnp.float32),
        pltpu.VMEM_SHARED((NP, D), jnp.float32),
        pltpu.SemaphoreType.DMA,
        pltpu.SemaphoreType.DMA,
    ],
)
def _sc_agg(g_hbm, src_hbm, dst_hbm, out_hbm, sidx_all, didx_all, rows0,
            rows1, acc_sh, sem0, sem1):
    cid = lax.axis_index("c")
    sid = lax.axis_index("s")
    wid = sid * NC + cid

    pltpu.async_copy(src_hbm.at[pl.ds(wid * EPW, EPW)], sidx_all, sem0)
    pltpu.async_copy(dst_hbm.at[wid], didx_all, sem1)

    def zero_row(r, _):
        for j in range(D // 16):
            rows0[r, pl.ds(j * 16, 16)] = jnp.zeros((16,), jnp.float32)
        return 0

    lax.fori_loop(0, CHUNK, zero_row, 0)
    for i in range(ROWS_PT // CHUNK):
        pltpu.sync_copy(rows0, acc_sh.at[pl.ds(sid * ROWS_PT + i * CHUNK, CHUNK)])
    pltpu.make_async_copy(src_hbm.at[pl.ds(wid * EPW, EPW)], sidx_all,
                          sem0).wait()
    pltpu.make_async_copy(dst_hbm.at[wid], didx_all, sem1).wait()
    plsc.subcore_barrier()

    def sidx(k):
        return sidx_all.at[pl.ds(k * CHUNK, CHUNK)]

    pltpu.async_copy(g_hbm.at[sidx(0)], rows0, sem0)

    def body(t, _):
        k0 = 2 * t
        pltpu.async_copy(g_hbm.at[sidx(k0 + 1)], rows1, sem1)
        pltpu.make_async_copy(g_hbm.at[sidx(k0)], rows0, sem0).wait()
        pltpu.sync_copy(rows0, acc_sh.at[didx_all.at[k0]], add=True)
        pltpu.async_copy(g_hbm.at[sidx(k0 + 2)], rows0, sem0)
        pltpu.make_async_copy(g_hbm.at[sidx(k0 + 1)], rows1, sem1).wait()
        pltpu.sync_copy(rows1, acc_sh.at[didx_all.at[k0 + 1]], add=True)
        return 0

    lax.fori_loop(0, (NCHUNK - 1) // 2, body, 0)
    pltpu.make_async_copy(g_hbm.at[sidx(NCHUNK - 1)], rows0, sem0).wait()
    pltpu.sync_copy(rows0, acc_sh.at[didx_all.at[NCHUNK - 1]], add=True)

    plsc.subcore_barrier()
    pltpu.sync_copy(acc_sh.at[pl.ds(sid * ROWS_PT, ROWS_PT)],
                    out_hbm.at[cid, pl.ds(sid * ROWS_PT, ROWS_PT)])


BLK = 1000


def _gemm_scale_body(x_ref, w_ref, dinv_ref, g_ref):
    h = jnp.dot(x_ref[...], w_ref[...], preferred_element_type=jnp.float32)
    g_ref[...] = h * dinv_ref[...]


def _tc_gemm_scale(x, w, dinv):
    return pl.pallas_call(
        _gemm_scale_body,
        grid=(N // BLK,),
        in_specs=[
            pl.BlockSpec((BLK, D), lambda i: (i, 0)),
            pl.BlockSpec((D, D), lambda i: (0, 0)),
            pl.BlockSpec((BLK, 1), lambda i: (i, 0)),
        ],
        out_specs=pl.BlockSpec((BLK, D), lambda i: (i, 0)),
        out_shape=jax.ShapeDtypeStruct((N, D), jnp.float32),
    )(x, w, dinv)


def _mid_body(p_ref, g_ref, dinv_ref, b_ref, w_ref, o_ref):
    acc = p_ref[0] + p_ref[1] + g_ref[...]
    x2 = jnp.maximum(acc * dinv_ref[...] + b_ref[...], 0.0)
    h2 = jnp.dot(x2, w_ref[...], preferred_element_type=jnp.float32)
    o_ref[...] = h2 * dinv_ref[...]


def _tc_mid(parts, g, dinv, b, w):
    return pl.pallas_call(
        _mid_body,
        grid=(N // BLK,),
        in_specs=[
            pl.BlockSpec((NC, BLK, D), lambda i: (0, i, 0)),
            pl.BlockSpec((BLK, D), lambda i: (i, 0)),
            pl.BlockSpec((BLK, 1), lambda i: (i, 0)),
            pl.BlockSpec((1, D), lambda i: (0, 0)),
            pl.BlockSpec((D, D), lambda i: (0, 0)),
        ],
        out_specs=pl.BlockSpec((BLK, D), lambda i: (i, 0)),
        out_shape=jax.ShapeDtypeStruct((N, D), jnp.float32),
    )(parts, g, dinv, b, w)


def _final_body(p_ref, g_ref, dinv_ref, b_ref, o_ref):
    acc = p_ref[0] + p_ref[1] + g_ref[...]
    o_ref[...] = acc * dinv_ref[...] + b_ref[...]


def _tc_final(parts, g, dinv, b):
    return pl.pallas_call(
        _final_body,
        grid=(N // BLK,),
        in_specs=[
            pl.BlockSpec((NC, BLK, D), lambda i: (0, i, 0)),
            pl.BlockSpec((BLK, D), lambda i: (i, 0)),
            pl.BlockSpec((BLK, 1), lambda i: (i, 0)),
            pl.BlockSpec((1, D), lambda i: (0, 0)),
        ],
        out_specs=pl.BlockSpec((BLK, D), lambda i: (i, 0)),
        out_shape=jax.ShapeDtypeStruct((N, D), jnp.float32),
    )(parts, g, dinv, b)


def kernel(edge_index, node_features, W1, b1, W2, b2):
    src = edge_index[0]
    dst = edge_index[1].reshape(NW, NCHUNK, CHUNK)

    degp = _sc_degree(dst)
    deg = degp[0, :N] + degp[1, :N] + 1.0
    dinv = lax.rsqrt(deg)[:, None]

    g1 = _tc_gemm_scale(node_features, W1, dinv)
    p1 = _sc_agg(g1, src, dst)
    g2 = _tc_mid(p1, g1, dinv, b1.reshape(1, D), W2)
    p2 = _sc_agg(g2, src, dst)
    return _tc_final(p2, g2, dinv, b2.reshape(1, D))
